# baseline (speedup 1.0000x reference)
import functools

import jax
import jax.numpy as jnp
from jax import lax
from jax.experimental import pallas as pl
from jax.experimental.pallas import tpu as pltpu
from jax.experimental.pallas import tpu_sc as plsc

N, E, NF, EF, L, TW, OUT, STEPS = 10000, 320000, 128, 4, 128, 5, 1, 5

NC, NS = 2, 16
NT = NC * NS
CH = 128
Np = 10240
ECH_T = 80
EPT = ECH_T * CH
Ep = NT * EPT
NPT = Np // NS
NCH_T = NPT // CH

F32 = jnp.float32
BF16 = jnp.bfloat16


@functools.cache
def _sc_kernels():
    mesh = plsc.VectorSubcoreMesh(
        core_axis_name="c", subcore_axis_name="s",
        num_cores=NC, num_subcores=NS)

    GCH = 64
    NCHUNK = EPT // GCH
    NPASS = 4
    CPP = NCHUNK // NPASS
    IPR = CPP * GCH // CH

    @functools.partial(
        pl.kernel,
        out_type=[jax.ShapeDtypeStruct((Ep, L), jnp.int32),
                  jax.ShapeDtypeStruct((Ep, L), jnp.int32)],
        mesh=mesh,
        scratch_types=[
            pltpu.VMEM((IPR, CH), jnp.int32),
            pltpu.VMEM((IPR, CH), jnp.int32),
            pltpu.VMEM((2, GCH, L), jnp.int32),
            pltpu.VMEM((2, GCH, L), jnp.int32),
            pltpu.VMEM_SHARED((Np, L), jnp.int32),
            pltpu.SemaphoreType.DMA,
            pltpu.SemaphoreType.DMA,
            pltpu.SemaphoreType.DMA,
            pltpu.SemaphoreType.DMA,
        ],
    )
    def sc_gather(pq_tab, s_idx, r_idx, gs_out, gr_out,
                  idx_s, idx_r, buf_s, buf_r, table, g0, g1, w0, w1):
        cid = lax.axis_index("c")
        sid = lax.axis_index("s")
        wid = sid * NC + cid
        base0 = wid * EPT
        gsem = (g0, g1)
        wsem = (w0, w1)

        row0 = sid * NPT

        def stage(t, carry):
            r0 = row0 + t * GCH
            pltpu.sync_copy(pq_tab.at[pl.ds(r0, GCH)], buf_s.at[0])
            pltpu.sync_copy(buf_s.at[0], table.at[pl.ds(r0, GCH)])
            return carry

        lax.fori_loop(0, NPT // GCH, stage, 0)
        plsc.subcore_barrier()

        def idx_slice(buf, c):
            return buf.at[c // 2, pl.ds((c % 2) * GCH, GCH)]

        def fire_g(slot, c):
            pltpu.async_copy(table.at[idx_slice(idx_s, c)], buf_s.at[slot],
                             gsem[slot])
            pltpu.async_copy(table.at[idx_slice(idx_r, c)], buf_r.at[slot],
                             gsem[slot])

        def wait_g(slot):
            pltpu.make_async_copy(table.at[idx_slice(idx_s, 0)],
                                  buf_s.at[slot], gsem[slot]).wait()
            pltpu.make_async_copy(table.at[idx_slice(idx_r, 0)],
                                  buf_r.at[slot], gsem[slot]).wait()

        def fire_w(slot, g):
            base = base0 + g * GCH
            pltpu.async_copy(buf_s.at[slot], gs_out.at[pl.ds(base, GCH)],
                             wsem[slot])
            pltpu.async_copy(buf_r.at[slot], gr_out.at[pl.ds(base, GCH)],
                             wsem[slot])

        def wait_w(slot):
            pltpu.make_async_copy(buf_s.at[slot], gs_out.at[pl.ds(0, GCH)],
                                  wsem[slot]).wait()
            pltpu.make_async_copy(buf_r.at[slot], gr_out.at[pl.ds(0, GCH)],
                                  wsem[slot]).wait()

        NHC = CPP // 2

        def run_pass(p):
            pltpu.sync_copy(s_idx.at[wid, p], idx_s)
            pltpu.sync_copy(r_idx.at[wid, p], idx_r)
            g_base = p * CPP
            fire_g(0, 0)

            def body(jj, carry):
                c0 = 2 * jj
                pl.when(jj > 0)(lambda: wait_w(1))
                fire_g(1, c0 + 1)
                wait_g(0)
                fire_w(0, g_base + c0)

                def refill_a():
                    wait_w(0)
                    fire_g(0, c0 + 2)

                pl.when(jj < NHC - 1)(refill_a)
                wait_g(1)
                fire_w(1, g_base + c0 + 1)
                return carry

            lax.fori_loop(0, NHC, body, 0)
            wait_w(0)
            wait_w(1)

        for p in range(NPASS):
            run_pass(p)

    @functools.partial(
        pl.kernel,
        out_type=jax.ShapeDtypeStruct((NC, Np, L), F32),
        mesh=mesh,
        scratch_types=[
            pltpu.VMEM((ECH_T, CH), jnp.int32),
            pltpu.VMEM((2, CH, L), F32),
            pltpu.VMEM_SHARED((Np, L), F32),
            pltpu.SemaphoreType.DMA,
            pltpu.SemaphoreType.DMA,
        ],
    )
    def sc_scatter(msgs, r_idx, zeros_rows, aggr_out, idx_v, rows, acc,
                   l0, l1):
        cid = lax.axis_index("c")
        sid = lax.axis_index("s")
        wid = sid * NC + cid

        pltpu.sync_copy(zeros_rows, rows.at[0])
        row0 = sid * NPT

        def zbody(t, carry):
            pltpu.sync_copy(rows.at[0], acc.at[pl.ds(row0 + t * CH, CH)])
            return carry

        lax.fori_loop(0, NCH_T, zbody, 0)
        pltpu.sync_copy(r_idx.at[wid], idx_v)
        plsc.subcore_barrier()

        base0 = wid * EPT
        lsem = (l0, l1)

        def fire_l(slot, j):
            pltpu.async_copy(msgs.at[pl.ds(base0 + j * CH, CH)],
                             rows.at[slot], lsem[slot])

        def wait_l(slot):
            pltpu.make_async_copy(msgs.at[pl.ds(0, CH)], rows.at[slot],
                                  lsem[slot]).wait()

        def scat(slot, j):
            pltpu.sync_copy(rows.at[slot], acc.at[idx_v.at[j]], add=True)

        fire_l(0, 0)
        NH = ECH_T // 2

        def body(jj, carry):
            j0 = 2 * jj
            fire_l(1, j0 + 1)
            wait_l(0)
            scat(0, j0)
            pl.when(jj < NH - 1)(lambda: fire_l(0, j0 + 2))
            wait_l(1)
            scat(1, j0 + 1)
            return carry

        lax.fori_loop(0, NH, body, 0)
        plsc.subcore_barrier()

        def dbody(t, carry):
            r0 = row0 + t * CH
            pltpu.sync_copy(acc.at[pl.ds(r0, CH)], rows.at[0])
            pltpu.sync_copy(rows.at[0], aggr_out.at[cid, pl.ds(r0, CH)])
            return carry

        lax.fori_loop(0, NCH_T, dbody, 0)

    return sc_gather, sc_scatter


def _ln(h, g, be):
    mu = jnp.mean(h, axis=-1, keepdims=True)
    var = jnp.mean((h - mu) ** 2, axis=-1, keepdims=True)
    return (h - mu) * lax.rsqrt(var + 1e-5) * g + be


def _row_spec(br, d):
    return pl.BlockSpec((br, d), lambda i: (i, 0))


def _full_spec(shape):
    return pl.BlockSpec(shape, lambda i: tuple(0 for _ in shape))


def _dot(a, b):
    return jnp.dot(a, b, preferred_element_type=F32)


def _pq(ln, e1a, e1b, po):
    po[...] = jnp.concatenate(
        [_dot(ln, e1a[...]), _dot(ln, e1b[...])], axis=-1).astype(BF16)


def _enc_node_body(x, m, s, w1, b1, w2, b2, g, be, e1a, e1b, o, po):
    xn = (x[...] - m[...]) / (s[...] + 1e-8)
    h = jnp.maximum(_dot(xn, w1[...]) + b1[...], 0.0)
    h = jnp.maximum(_dot(h, w2[...]) + b2[...], 0.0)
    ln = _ln(h, g[...], be[...])
    o[...] = ln
    _pq(ln, e1a, e1b, po)


def _enc_edge(x, m, s, w1, b1, w2, b2, g, be):
    xn = (x[...] - m[...]) / (s[...] + 1e-8)
    h = b1[...]
    for f in range(EF):
        h = h + xn[:, f:f + 1] * w1[f:f + 1, :]
    h = jnp.maximum(h, 0.0)
    h = jnp.maximum(_dot(h, w2[...]) + b2[...], 0.0)
    return _ln(h, g[...], be[...])


def _edge_core(gs2, gr2, el, w1c, b1, w2, b2, g, be):
    ps = gs2[:, :L].astype(F32)
    qr = gr2[:, L:].astype(F32)
    x = ps + qr + _dot(el, w1c[...]) + b1[...]
    h = jnp.maximum(x, 0.0)
    h = jnp.maximum(_dot(h, w2[...]) + b2[...], 0.0)
    return _ln(h, g[...], be[...])


def _edge_first_body(gs2, gr2, ef, em, es, ew1, eb1, ew2, eb2, eg, ebe,
                     w1c, b1, w2, b2, g, be, ne, nl):
    el = _enc_edge(ef, em, es, ew1, eb1, ew2, eb2, eg, ebe)
    ln = _edge_core(gs2[...], gr2[...], el, w1c, b1, w2, b2, g, be)
    ne[...] = ln
    nl[...] = el + ln


def _edge_mid_body(gs2, gr2, el, w1c, b1, w2, b2, g, be, ne, nl):
    ln = _edge_core(gs2[...], gr2[...], el[...], w1c, b1, w2, b2, g, be)
    ne[...] = ln
    nl[...] = el[...] + ln


def _edge_last_body(gs2, gr2, el, w1c, b1, w2, b2, g, be, ne):
    ne[...] = _edge_core(gs2[...], gr2[...], el[...], w1c, b1, w2, b2, g, be)


def _node_core(nl, a0, a1, w1a, w1b, b1, w2, b2, g, be):
    x = (_dot(nl[...], w1a[...]) + _dot(a0[...] + a1[...], w1b[...]) + b1[...])
    h = jnp.maximum(x, 0.0)
    h = jnp.maximum(_dot(h, w2[...]) + b2[...], 0.0)
    return _ln(h, g[...], be[...]) + nl[...]


def _node_step_pq_body(nl, a0, a1, w1a, w1b, b1, w2, b2, g, be, e1a, e1b,
                       o, po):
    out = _node_core(nl, a0, a1, w1a, w1b, b1, w2, b2, g, be)
    o[...] = out
    _pq(out, e1a, e1b, po)


def _node_last_body(nl, a0, a1, w1a, w1b, b1, w2, b2, g, be, o):
    o[...] = _node_core(nl, a0, a1, w1a, w1b, b1, w2, b2, g, be)


def _decoder_body(x, w1p, b1p, w2p, b2p, dt, o):
    h = _dot(x[...], w1p[...]) + b1p[...]
    h = h * jax.nn.sigmoid(h)
    o[...] = (_dot(h, w2p[...]) + b2p[...]) * dt[...]


def _mlp_call(body, rows, br, in_feats, outs, weight_shapes):
    grid = rows // br
    in_specs = ([_row_spec(br, d) for d, _ in in_feats]
                + [_full_spec(s) for s in weight_shapes])
    return pl.pallas_call(
        body,
        grid=(grid,),
        in_specs=in_specs,
        out_specs=[_row_spec(br, d) for d, _ in outs],
        out_shape=[jax.ShapeDtypeStruct((rows, d), dt) for d, dt in outs],
        compiler_params=pltpu.CompilerParams(
            dimension_semantics=("parallel",)),
    )


_W = (L, L)
_B = (1, L)
_F = [(L, F32)]
_H = [(L, BF16)]
_PQO = [(L, F32), (2 * L, BF16)]
_EDGE_W = [_W, _B, _W, _B, _B, _B]
_ENC_W = [(1, EF), (1, EF), (EF, L), _B, _W, _B, _B, _B]

_H2 = [(2 * L, BF16)] * 2

_enc_node = _mlp_call(_enc_node_body, Np, 2560, [(NF, F32)],
                      _PQO, [_B, _B, _W, _B, _W, _B, _B, _B, _W, _W])
_edge_first = _mlp_call(_edge_first_body, Ep, 4096, _H2 + [(EF, F32)],
                        _F * 2, _ENC_W + _EDGE_W)
_edge_mid = _mlp_call(_edge_mid_body, Ep, 4096, _H2 + _F,
                      _F * 2, _EDGE_W)
_edge_last = _mlp_call(_edge_last_body, Ep, 4096, _H2 + _F,
                       _F, _EDGE_W)
_node_step_pq = _mlp_call(_node_step_pq_body, Np, 2560, _F * 3,
                          _PQO, [_W, _W, _B, _W, _B, _B, _B, _W, _W])
_node_last = _mlp_call(_node_last_body, Np, 2560, _F * 3,
                       _F, [_W, _W, _B, _W, _B, _B, _B])
_decoder = _mlp_call(_decoder_body, Np, 2560, _F,
                     _F, [_W, _B, _W, _B, _B])


def kernel(node_features, mesh_edge_features, senders, receivers, params):
    p = params
    nf = jnp.pad(node_features.reshape(N, NF), ((0, Np - N), (0, 0)))
    ef = jnp.pad(mesh_edge_features.reshape(E, EF), ((0, Ep - E), (0, 0)))
    s_pad = jnp.pad(senders.astype(jnp.int32), (0, Ep - E),
                    constant_values=Np - 1)
    r_pad = jnp.pad(receivers.astype(jnp.int32), (0, Ep - E),
                    constant_values=Np - 1)
    s_idx4 = s_pad.reshape(NT, 4, EPT // (4 * CH), CH)
    r_idx4 = r_pad.reshape(NT, 4, EPT // (4 * CH), CH)
    r_idx3 = r_pad.reshape(NT, ECH_T, CH)

    def row(v):
        return v.reshape(1, -1)

    def ew1(k):
        w1 = p["blk%d_e_W1" % k]
        return w1[:L], w1[L:2 * L], w1[2 * L:]

    def edge_w(k):
        pe = "blk%d_e" % k
        return (ew1(k)[2], row(p[pe + "_b1"]), p[pe + "_W2"],
                row(p[pe + "_b2"]), row(p[pe + "_g"]), row(p[pe + "_be"]))

    def node_w(k):
        pn = "blk%d_n" % k
        w1n = p[pn + "_W1"]
        return (w1n[:L], w1n[L:], row(p[pn + "_b1"]), p[pn + "_W2"],
                row(p[pn + "_b2"]), row(p[pn + "_g"]), row(p[pn + "_be"]))

    e1a0, e1b0, _ = ew1(0)
    node_lat, pq = _enc_node(
        nf, row(p["node_mean"]), row(p["node_std"]),
        p["enc_n_W1"], row(p["enc_n_b1"]), p["enc_n_W2"], row(p["enc_n_b2"]),
        row(p["enc_n_g"]), row(p["enc_n_be"]), e1a0, e1b0)

    sc_gather, sc_scatter = _sc_kernels()
    zeros_rows = jnp.zeros((CH, L), F32)

    def to_i32(x):
        return lax.bitcast_convert_type(
            x.reshape(x.shape[0], L, 2), jnp.int32)

    def to_bf16(x):
        return lax.bitcast_convert_type(x, BF16).reshape(-1, 2 * L)

    edge_lat = None
    for k in range(STEPS):
        gs_i, gr_i = sc_gather(to_i32(pq), s_idx4, r_idx4)
        gs2, gr2 = to_bf16(gs_i), to_bf16(gr_i)
        if k == 0:
            new_e, edge_lat = _edge_first(
                gs2, gr2, ef, row(p["edge_mean"]), row(p["edge_std"]),
                p["enc_e_W1"], row(p["enc_e_b1"]), p["enc_e_W2"],
                row(p["enc_e_b2"]), row(p["enc_e_g"]), row(p["enc_e_be"]),
                *edge_w(k))
        elif k < STEPS - 1:
            new_e, edge_lat = _edge_mid(gs2, gr2, edge_lat, *edge_w(k))
        else:
            (new_e,) = _edge_last(gs2, gr2, edge_lat, *edge_w(k))
        aggr = sc_scatter(new_e, r_idx3, zeros_rows)
        if k < STEPS - 1:
            e1a, e1b, _ = ew1(k + 1)
            node_lat, pq = _node_step_pq(
                node_lat, aggr[0], aggr[1], *node_w(k), e1a, e1b)
        else:
            (node_lat,) = _node_last(node_lat, aggr[0], aggr[1], *node_w(k))

    w1p = jnp.zeros((L, L), F32).at[:, :8].set(p["dec_W1"].T)
    b1p = jnp.zeros((1, L), F32).at[0, :8].set(p["dec_b1"])
    w2p = jnp.zeros((L, L), F32).at[:8, :TW].set(p["dec_W2"].T)
    b2p = jnp.zeros((1, L), F32).at[0, :TW].set(p["dec_b2"])
    dt = jnp.zeros((1, L), F32).at[0, :TW].set(
        jnp.arange(1, TW + 1, dtype=F32))
    (dec,) = _decoder(node_lat, w1p, b1p, w2p, b2p, dt)
    return dec[:N, :TW].T[:, :, None]


# speedup vs baseline: 1.0042x; 1.0042x over previous
import functools

import jax
import jax.numpy as jnp
from jax import lax
from jax.experimental import pallas as pl
from jax.experimental.pallas import tpu as pltpu
from jax.experimental.pallas import tpu_sc as plsc

N, E, NF, EF, L, TW, OUT, STEPS = 10000, 320000, 128, 4, 128, 5, 1, 5

NC, NS = 2, 16
NT = NC * NS
CH = 128
Np = 10240
ECH_T = 80
EPT = ECH_T * CH
Ep = NT * EPT
NPT = Np // NS
NCH_T = NPT // CH

F32 = jnp.float32
BF16 = jnp.bfloat16


@functools.cache
def _sc_kernels():
    mesh = plsc.VectorSubcoreMesh(
        core_axis_name="c", subcore_axis_name="s",
        num_cores=NC, num_subcores=NS)

    @functools.partial(
        pl.kernel,
        out_type=[jax.ShapeDtypeStruct((Ep, L), jnp.int32),
                  jax.ShapeDtypeStruct((Ep, L), jnp.int32)],
        mesh=mesh,
        scratch_types=[
            pltpu.VMEM((ECH_T, CH), jnp.int32),
            pltpu.VMEM((2, CH, L), jnp.int32),
            pltpu.VMEM_SHARED((Np, L), jnp.int32),
            pltpu.SemaphoreType.DMA,
            pltpu.SemaphoreType.DMA,
            pltpu.SemaphoreType.DMA,
            pltpu.SemaphoreType.DMA,
        ],
    )
    def sc_gather(pq_tab, s_idx, r_idx, gs_out, gr_out,
                  idx, bufs, table, g0, g1, w0, w1):
        cid = lax.axis_index("c")
        sid = lax.axis_index("s")
        wid = sid * NC + cid
        base0 = wid * EPT
        gsem = (g0, g1)
        wsem = (w0, w1)

        row0 = sid * NPT

        def stage(t, carry):
            r0 = row0 + t * CH
            pltpu.sync_copy(pq_tab.at[pl.ds(r0, CH)], bufs.at[0])
            pltpu.sync_copy(bufs.at[0], table.at[pl.ds(r0, CH)])
            return carry

        lax.fori_loop(0, NCH_T, stage, 0)
        pltpu.sync_copy(s_idx.at[wid], idx)
        plsc.subcore_barrier()

        NH = ECH_T // 2

        def run_phase(out_ref):

            def fire_g(slot, j):
                pltpu.async_copy(table.at[idx.at[j]], bufs.at[slot],
                                 gsem[slot])

            def wait_g(slot):
                pltpu.make_async_copy(table.at[idx.at[0]], bufs.at[slot],
                                      gsem[slot]).wait()

            def fire_w(slot, j):
                pltpu.async_copy(bufs.at[slot],
                                 out_ref.at[pl.ds(base0 + j * CH, CH)],
                                 wsem[slot])

            def wait_w(slot):
                pltpu.make_async_copy(bufs.at[slot],
                                      out_ref.at[pl.ds(0, CH)],
                                      wsem[slot]).wait()

            fire_g(0, 0)

            def body(jj, carry):
                j0 = 2 * jj
                pl.when(jj > 0)(lambda: wait_w(1))
                fire_g(1, j0 + 1)
                wait_g(0)
                fire_w(0, j0)

                def refill_a():
                    wait_w(0)
                    fire_g(0, j0 + 2)

                pl.when(jj < NH - 1)(refill_a)
                wait_g(1)
                fire_w(1, j0 + 1)
                return carry

            lax.fori_loop(0, NH, body, 0)
            wait_w(0)
            wait_w(1)

        run_phase(gs_out)
        pltpu.sync_copy(r_idx.at[wid], idx)
        run_phase(gr_out)

    @functools.partial(
        pl.kernel,
        out_type=jax.ShapeDtypeStruct((NC, Np, L), F32),
        mesh=mesh,
        scratch_types=[
            pltpu.VMEM((ECH_T, CH), jnp.int32),
            pltpu.VMEM((2, CH, L), F32),
            pltpu.VMEM_SHARED((Np, L), F32),
            pltpu.SemaphoreType.DMA,
            pltpu.SemaphoreType.DMA,
        ],
    )
    def sc_scatter(msgs, r_idx, zeros_rows, aggr_out, idx_v, rows, acc,
                   l0, l1):
        cid = lax.axis_index("c")
        sid = lax.axis_index("s")
        wid = sid * NC + cid

        pltpu.sync_copy(zeros_rows, rows.at[0])
        row0 = sid * NPT

        def zbody(t, carry):
            pltpu.sync_copy(rows.at[0], acc.at[pl.ds(row0 + t * CH, CH)])
            return carry

        lax.fori_loop(0, NCH_T, zbody, 0)
        pltpu.sync_copy(r_idx.at[wid], idx_v)
        plsc.subcore_barrier()

        base0 = wid * EPT
        lsem = (l0, l1)

        def fire_l(slot, j):
            pltpu.async_copy(msgs.at[pl.ds(base0 + j * CH, CH)],
                             rows.at[slot], lsem[slot])

        def wait_l(slot):
            pltpu.make_async_copy(msgs.at[pl.ds(0, CH)], rows.at[slot],
                                  lsem[slot]).wait()

        def scat(slot, j):
            pltpu.sync_copy(rows.at[slot], acc.at[idx_v.at[j]], add=True)

        fire_l(0, 0)
        NH = ECH_T // 2

        def body(jj, carry):
            j0 = 2 * jj
            fire_l(1, j0 + 1)
            wait_l(0)
            scat(0, j0)
            pl.when(jj < NH - 1)(lambda: fire_l(0, j0 + 2))
            wait_l(1)
            scat(1, j0 + 1)
            return carry

        lax.fori_loop(0, NH, body, 0)
        plsc.subcore_barrier()

        def dbody(t, carry):
            r0 = row0 + t * CH
            pltpu.sync_copy(acc.at[pl.ds(r0, CH)], rows.at[0])
            pltpu.sync_copy(rows.at[0], aggr_out.at[cid, pl.ds(r0, CH)])
            return carry

        lax.fori_loop(0, NCH_T, dbody, 0)

    return sc_gather, sc_scatter


def _ln(h, g, be):
    mu = jnp.mean(h, axis=-1, keepdims=True)
    var = jnp.mean((h - mu) ** 2, axis=-1, keepdims=True)
    return (h - mu) * lax.rsqrt(var + 1e-5) * g + be


def _row_spec(br, d):
    return pl.BlockSpec((br, d), lambda i: (i, 0))


def _full_spec(shape):
    return pl.BlockSpec(shape, lambda i: tuple(0 for _ in shape))


def _dot(a, b):
    return jnp.dot(a, b, preferred_element_type=F32)


def _pq(ln, e1a, e1b, po):
    po[...] = jnp.concatenate(
        [_dot(ln, e1a[...]), _dot(ln, e1b[...])], axis=-1).astype(BF16)


def _enc_node_body(x, m, s, w1, b1, w2, b2, g, be, e1a, e1b, o, po):
    xn = (x[...] - m[...]) / (s[...] + 1e-8)
    h = jnp.maximum(_dot(xn, w1[...]) + b1[...], 0.0)
    h = jnp.maximum(_dot(h, w2[...]) + b2[...], 0.0)
    ln = _ln(h, g[...], be[...])
    o[...] = ln
    _pq(ln, e1a, e1b, po)


def _enc_edge(x, m, s, w1, b1, w2, b2, g, be):
    xn = (x[...] - m[...]) / (s[...] + 1e-8)
    h = b1[...]
    for f in range(EF):
        h = h + xn[:, f:f + 1] * w1[f:f + 1, :]
    h = jnp.maximum(h, 0.0)
    h = jnp.maximum(_dot(h, w2[...]) + b2[...], 0.0)
    return _ln(h, g[...], be[...])


def _edge_core(gs2, gr2, el, w1c, b1, w2, b2, g, be):
    ps = gs2[:, :L].astype(F32)
    qr = gr2[:, L:].astype(F32)
    x = ps + qr + _dot(el, w1c[...]) + b1[...]
    h = jnp.maximum(x, 0.0)
    h = jnp.maximum(_dot(h, w2[...]) + b2[...], 0.0)
    return _ln(h, g[...], be[...])


def _edge_first_body(gs2, gr2, ef, em, es, ew1, eb1, ew2, eb2, eg, ebe,
                     w1c, b1, w2, b2, g, be, ne, nl):
    el = _enc_edge(ef, em, es, ew1, eb1, ew2, eb2, eg, ebe)
    ln = _edge_core(gs2[...], gr2[...], el, w1c, b1, w2, b2, g, be)
    ne[...] = ln
    nl[...] = el + ln


def _edge_mid_body(gs2, gr2, el, w1c, b1, w2, b2, g, be, ne, nl):
    ln = _edge_core(gs2[...], gr2[...], el[...], w1c, b1, w2, b2, g, be)
    ne[...] = ln
    nl[...] = el[...] + ln


def _edge_last_body(gs2, gr2, el, w1c, b1, w2, b2, g, be, ne):
    ne[...] = _edge_core(gs2[...], gr2[...], el[...], w1c, b1, w2, b2, g, be)


def _node_core(nl, a0, a1, w1a, w1b, b1, w2, b2, g, be):
    x = (_dot(nl[...], w1a[...]) + _dot(a0[...] + a1[...], w1b[...]) + b1[...])
    h = jnp.maximum(x, 0.0)
    h = jnp.maximum(_dot(h, w2[...]) + b2[...], 0.0)
    return _ln(h, g[...], be[...]) + nl[...]


def _node_step_pq_body(nl, a0, a1, w1a, w1b, b1, w2, b2, g, be, e1a, e1b,
                       o, po):
    out = _node_core(nl, a0, a1, w1a, w1b, b1, w2, b2, g, be)
    o[...] = out
    _pq(out, e1a, e1b, po)


def _node_last_body(nl, a0, a1, w1a, w1b, b1, w2, b2, g, be, o):
    o[...] = _node_core(nl, a0, a1, w1a, w1b, b1, w2, b2, g, be)


def _decoder_body(x, w1p, b1p, w2p, b2p, dt, o):
    h = _dot(x[...], w1p[...]) + b1p[...]
    h = h * jax.nn.sigmoid(h)
    o[...] = (_dot(h, w2p[...]) + b2p[...]) * dt[...]


def _mlp_call(body, rows, br, in_feats, outs, weight_shapes):
    grid = rows // br
    in_specs = ([_row_spec(br, d) for d, _ in in_feats]
                + [_full_spec(s) for s in weight_shapes])
    return pl.pallas_call(
        body,
        grid=(grid,),
        in_specs=in_specs,
        out_specs=[_row_spec(br, d) for d, _ in outs],
        out_shape=[jax.ShapeDtypeStruct((rows, d), dt) for d, dt in outs],
        compiler_params=pltpu.CompilerParams(
            dimension_semantics=("parallel",)),
    )


_W = (L, L)
_B = (1, L)
_F = [(L, F32)]
_H = [(L, BF16)]
_PQO = [(L, F32), (2 * L, BF16)]
_EDGE_W = [_W, _B, _W, _B, _B, _B]
_ENC_W = [(1, EF), (1, EF), (EF, L), _B, _W, _B, _B, _B]

_H2 = [(2 * L, BF16)] * 2

_enc_node = _mlp_call(_enc_node_body, Np, 2560, [(NF, F32)],
                      _PQO, [_B, _B, _W, _B, _W, _B, _B, _B, _W, _W])
_edge_first = _mlp_call(_edge_first_body, Ep, 4096, _H2 + [(EF, F32)],
                        _F * 2, _ENC_W + _EDGE_W)
_edge_mid = _mlp_call(_edge_mid_body, Ep, 4096, _H2 + _F,
                      _F * 2, _EDGE_W)
_edge_last = _mlp_call(_edge_last_body, Ep, 4096, _H2 + _F,
                       _F, _EDGE_W)
_node_step_pq = _mlp_call(_node_step_pq_body, Np, 2560, _F * 3,
                          _PQO, [_W, _W, _B, _W, _B, _B, _B, _W, _W])
_node_last = _mlp_call(_node_last_body, Np, 2560, _F * 3,
                       _F, [_W, _W, _B, _W, _B, _B, _B])
_decoder = _mlp_call(_decoder_body, Np, 2560, _F,
                     _F, [_W, _B, _W, _B, _B])


def kernel(node_features, mesh_edge_features, senders, receivers, params):
    p = params
    nf = jnp.pad(node_features.reshape(N, NF), ((0, Np - N), (0, 0)))
    ef = jnp.pad(mesh_edge_features.reshape(E, EF), ((0, Ep - E), (0, 0)))
    s_pad = jnp.pad(senders.astype(jnp.int32), (0, Ep - E),
                    constant_values=Np - 1)
    r_pad = jnp.pad(receivers.astype(jnp.int32), (0, Ep - E),
                    constant_values=Np - 1)
    s_idx3 = s_pad.reshape(NT, ECH_T, CH)
    r_idx3 = r_pad.reshape(NT, ECH_T, CH)

    def row(v):
        return v.reshape(1, -1)

    def ew1(k):
        w1 = p["blk%d_e_W1" % k]
        return w1[:L], w1[L:2 * L], w1[2 * L:]

    def edge_w(k):
        pe = "blk%d_e" % k
        return (ew1(k)[2], row(p[pe + "_b1"]), p[pe + "_W2"],
                row(p[pe + "_b2"]), row(p[pe + "_g"]), row(p[pe + "_be"]))

    def node_w(k):
        pn = "blk%d_n" % k
        w1n = p[pn + "_W1"]
        return (w1n[:L], w1n[L:], row(p[pn + "_b1"]), p[pn + "_W2"],
                row(p[pn + "_b2"]), row(p[pn + "_g"]), row(p[pn + "_be"]))

    e1a0, e1b0, _ = ew1(0)
    node_lat, pq = _enc_node(
        nf, row(p["node_mean"]), row(p["node_std"]),
        p["enc_n_W1"], row(p["enc_n_b1"]), p["enc_n_W2"], row(p["enc_n_b2"]),
        row(p["enc_n_g"]), row(p["enc_n_be"]), e1a0, e1b0)

    sc_gather, sc_scatter = _sc_kernels()
    zeros_rows = jnp.zeros((CH, L), F32)

    def to_i32(x):
        return lax.bitcast_convert_type(
            x.reshape(x.shape[0], L, 2), jnp.int32)

    def to_bf16(x):
        return lax.bitcast_convert_type(x, BF16).reshape(-1, 2 * L)

    edge_lat = None
    for k in range(STEPS):
        gs_i, gr_i = sc_gather(to_i32(pq), s_idx3, r_idx3)
        gs2, gr2 = to_bf16(gs_i), to_bf16(gr_i)
        if k == 0:
            new_e, edge_lat = _edge_first(
                gs2, gr2, ef, row(p["edge_mean"]), row(p["edge_std"]),
                p["enc_e_W1"], row(p["enc_e_b1"]), p["enc_e_W2"],
                row(p["enc_e_b2"]), row(p["enc_e_g"]), row(p["enc_e_be"]),
                *edge_w(k))
        elif k < STEPS - 1:
            new_e, edge_lat = _edge_mid(gs2, gr2, edge_lat, *edge_w(k))
        else:
            (new_e,) = _edge_last(gs2, gr2, edge_lat, *edge_w(k))
        aggr = sc_scatter(new_e, r_idx3, zeros_rows)
        if k < STEPS - 1:
            e1a, e1b, _ = ew1(k + 1)
            node_lat, pq = _node_step_pq(
                node_lat, aggr[0], aggr[1], *node_w(k), e1a, e1b)
        else:
            (node_lat,) = _node_last(node_lat, aggr[0], aggr[1], *node_w(k))

    w1p = jnp.zeros((L, L), F32).at[:, :8].set(p["dec_W1"].T)
    b1p = jnp.zeros((1, L), F32).at[0, :8].set(p["dec_b1"])
    w2p = jnp.zeros((L, L), F32).at[:8, :TW].set(p["dec_W2"].T)
    b2p = jnp.zeros((1, L), F32).at[0, :TW].set(p["dec_b2"])
    dt = jnp.zeros((1, L), F32).at[0, :TW].set(
        jnp.arange(1, TW + 1, dtype=F32))
    (dec,) = _decoder(node_lat, w1p, b1p, w2p, b2p, dt)
    return dec[:N, :TW].T[:, :, None]


# speedup vs baseline: 5.5264x; 5.5034x over previous
import functools

import jax
import jax.numpy as jnp
from jax import lax
from jax.experimental import pallas as pl
from jax.experimental.pallas import tpu as pltpu
from jax.experimental.pallas import tpu_sc as plsc

N, E, NF, EF, L, TW, OUT, STEPS = 10000, 320000, 128, 4, 128, 5, 1, 5

NC, NS = 2, 16
NT = NC * NS
CH = 128
Np = 10240
ECH_T = 80
EPT = ECH_T * CH
Ep = NT * EPT
NPT = Np // NS
NCH_T = NPT // CH

F32 = jnp.float32
BF16 = jnp.bfloat16


@functools.cache
def _sc_kernels():
    mesh = plsc.VectorSubcoreMesh(
        core_axis_name="c", subcore_axis_name="s",
        num_cores=NC, num_subcores=NS)

    @functools.partial(
        pl.kernel,
        out_type=[jax.ShapeDtypeStruct((Ep, L), F32),
                  jax.ShapeDtypeStruct((Ep, L), F32)],
        mesh=mesh,
        scratch_types=[
            pltpu.VMEM((ECH_T, CH), jnp.int32),
            pltpu.VMEM((2, CH, L), F32),
            pltpu.VMEM_SHARED((Np, L), F32),
            pltpu.SemaphoreType.DMA,
            pltpu.SemaphoreType.DMA,
            pltpu.SemaphoreType.DMA,
            pltpu.SemaphoreType.DMA,
        ],
    )
    def sc_gather(nodes, s_idx, r_idx, gs_out, gr_out,
                  idx, bufs, table, g0, g1, w0, w1):
        cid = lax.axis_index("c")
        sid = lax.axis_index("s")
        wid = sid * NC + cid
        base0 = wid * EPT
        gsem = (g0, g1)
        wsem = (w0, w1)

        row0 = sid * NPT

        def stage(t, carry):
            r0 = row0 + t * CH
            pltpu.sync_copy(nodes.at[pl.ds(r0, CH)], bufs.at[0])
            pltpu.sync_copy(bufs.at[0], table.at[pl.ds(r0, CH)])
            return carry

        lax.fori_loop(0, NCH_T, stage, 0)
        pltpu.sync_copy(s_idx.at[wid], idx)
        plsc.subcore_barrier()

        NH = ECH_T // 2

        def run_phase(out_ref):

            def fire_g(slot, j):
                pltpu.async_copy(table.at[idx.at[j]], bufs.at[slot],
                                 gsem[slot])

            def wait_g(slot):
                pltpu.make_async_copy(table.at[idx.at[0]], bufs.at[slot],
                                      gsem[slot]).wait()

            def fire_w(slot, j):
                pltpu.async_copy(bufs.at[slot],
                                 out_ref.at[pl.ds(base0 + j * CH, CH)],
                                 wsem[slot])

            def wait_w(slot):
                pltpu.make_async_copy(bufs.at[slot],
                                      out_ref.at[pl.ds(0, CH)],
                                      wsem[slot]).wait()

            fire_g(0, 0)

            def body(jj, carry):
                j0 = 2 * jj
                pl.when(jj > 0)(lambda: wait_w(1))
                fire_g(1, j0 + 1)
                wait_g(0)
                fire_w(0, j0)

                def refill_a():
                    wait_w(0)
                    fire_g(0, j0 + 2)

                pl.when(jj < NH - 1)(refill_a)
                wait_g(1)
                fire_w(1, j0 + 1)
                return carry

            lax.fori_loop(0, NH, body, 0)
            wait_w(0)
            wait_w(1)

        run_phase(gs_out)
        pltpu.sync_copy(r_idx.at[wid], idx)
        run_phase(gr_out)

    @functools.partial(
        pl.kernel,
        out_type=jax.ShapeDtypeStruct((NC, Np, L), F32),
        mesh=mesh,
        scratch_types=[
            pltpu.VMEM((ECH_T, CH), jnp.int32),
            pltpu.VMEM((2, CH, L), F32),
            pltpu.VMEM_SHARED((Np, L), F32),
            pltpu.SemaphoreType.DMA,
            pltpu.SemaphoreType.DMA,
        ],
    )
    def sc_scatter(msgs, r_idx, zeros_rows, aggr_out, idx_v, rows, acc,
                   l0, l1):
        cid = lax.axis_index("c")
        sid = lax.axis_index("s")
        wid = sid * NC + cid

        pltpu.sync_copy(zeros_rows, rows.at[0])
        row0 = sid * NPT

        def zbody(t, carry):
            pltpu.sync_copy(rows.at[0], acc.at[pl.ds(row0 + t * CH, CH)])
            return carry

        lax.fori_loop(0, NCH_T, zbody, 0)
        pltpu.sync_copy(r_idx.at[wid], idx_v)
        plsc.subcore_barrier()

        base0 = wid * EPT
        lsem = (l0, l1)

        def fire_l(slot, j):
            pltpu.async_copy(msgs.at[pl.ds(base0 + j * CH, CH)],
                             rows.at[slot], lsem[slot])

        def wait_l(slot):
            pltpu.make_async_copy(msgs.at[pl.ds(0, CH)], rows.at[slot],
                                  lsem[slot]).wait()

        def scat(slot, j):
            pltpu.sync_copy(rows.at[slot], acc.at[idx_v.at[j]], add=True)

        fire_l(0, 0)
        NH = ECH_T // 2

        def body(jj, carry):
            j0 = 2 * jj
            fire_l(1, j0 + 1)
            wait_l(0)
            scat(0, j0)
            pl.when(jj < NH - 1)(lambda: fire_l(0, j0 + 2))
            wait_l(1)
            scat(1, j0 + 1)
            return carry

        lax.fori_loop(0, NH, body, 0)
        plsc.subcore_barrier()

        def dbody(t, carry):
            r0 = row0 + t * CH
            pltpu.sync_copy(acc.at[pl.ds(r0, CH)], rows.at[0])
            pltpu.sync_copy(rows.at[0], aggr_out.at[cid, pl.ds(r0, CH)])
            return carry

        lax.fori_loop(0, NCH_T, dbody, 0)

    return sc_gather, sc_scatter


def _ln(h, g, be):
    mu = jnp.mean(h, axis=-1, keepdims=True)
    var = jnp.mean((h - mu) ** 2, axis=-1, keepdims=True)
    return (h - mu) * lax.rsqrt(var + 1e-5) * g + be


def _row_spec(br, d):
    return pl.BlockSpec((br, d), lambda i: (i, 0))


def _full_spec(shape):
    return pl.BlockSpec(shape, lambda i: tuple(0 for _ in shape))


def _dot(a, b):
    return jnp.dot(a, b, preferred_element_type=F32)


def _enc_node_body(x, m, s, w1, b1, w2, b2, g, be, o):
    xn = (x[...] - m[...]) / (s[...] + 1e-8)
    h = jnp.maximum(_dot(xn, w1[...]) + b1[...], 0.0)
    h = jnp.maximum(_dot(h, w2[...]) + b2[...], 0.0)
    o[...] = _ln(h, g[...], be[...])


def _enc_edge(x, m, s, w1, b1, w2, b2, g, be):
    xn = (x[...] - m[...]) / (s[...] + 1e-8)
    h = b1[...]
    for f in range(EF):
        h = h + xn[:, f:f + 1] * w1[f:f + 1, :]
    h = jnp.maximum(h, 0.0)
    h = jnp.maximum(_dot(h, w2[...]) + b2[...], 0.0)
    return _ln(h, g[...], be[...])


def _edge_core(gs, gr, el, w1a, w1b, w1c, b1, w2, b2, g, be):
    x = (_dot(gs, w1a[...]) + _dot(gr, w1b[...])
         + _dot(el, w1c[...]) + b1[...])
    h = jnp.maximum(x, 0.0)
    h = jnp.maximum(_dot(h, w2[...]) + b2[...], 0.0)
    return _ln(h, g[...], be[...])


def _edge_first_body(gs, gr, ef, em, es, ew1, eb1, ew2, eb2, eg, ebe,
                     w1a, w1b, w1c, b1, w2, b2, g, be, ne, nl):
    el = _enc_edge(ef, em, es, ew1, eb1, ew2, eb2, eg, ebe)
    ln = _edge_core(gs[...], gr[...], el, w1a, w1b, w1c, b1, w2, b2, g, be)
    ne[...] = ln
    nl[...] = el + ln


def _edge_mid_body(gs, gr, el, w1a, w1b, w1c, b1, w2, b2, g, be, ne, nl):
    ln = _edge_core(gs[...], gr[...], el[...], w1a, w1b, w1c, b1, w2, b2,
                    g, be)
    ne[...] = ln
    nl[...] = el[...] + ln


def _edge_last_body(gs, gr, el, w1a, w1b, w1c, b1, w2, b2, g, be, ne):
    ne[...] = _edge_core(gs[...], gr[...], el[...], w1a, w1b, w1c, b1, w2, b2,
                         g, be)


def _node_core(nl, a0, a1, w1a, w1b, b1, w2, b2, g, be):
    x = (_dot(nl[...], w1a[...]) + _dot(a0[...] + a1[...], w1b[...]) + b1[...])
    h = jnp.maximum(x, 0.0)
    h = jnp.maximum(_dot(h, w2[...]) + b2[...], 0.0)
    return _ln(h, g[...], be[...]) + nl[...]


def _node_last_body(nl, a0, a1, w1a, w1b, b1, w2, b2, g, be, o):
    o[...] = _node_core(nl, a0, a1, w1a, w1b, b1, w2, b2, g, be)


def _decoder_body(x, w1p, b1p, w2p, b2p, dt, o):
    h = _dot(x[...], w1p[...]) + b1p[...]
    h = h * jax.nn.sigmoid(h)
    o[...] = (_dot(h, w2p[...]) + b2p[...]) * dt[...]


def _mlp_call(body, rows, br, in_feats, outs, weight_shapes):
    grid = rows // br
    in_specs = ([_row_spec(br, d) for d, _ in in_feats]
                + [_full_spec(s) for s in weight_shapes])
    return pl.pallas_call(
        body,
        grid=(grid,),
        in_specs=in_specs,
        out_specs=[_row_spec(br, d) for d, _ in outs],
        out_shape=[jax.ShapeDtypeStruct((rows, d), dt) for d, dt in outs],
        compiler_params=pltpu.CompilerParams(
            dimension_semantics=("parallel",)),
    )


_W = (L, L)
_B = (1, L)
_F = [(L, F32)]
_H = [(L, BF16)]
_EDGE_W = [_W, _W, _W, _B, _W, _B, _B, _B]
_ENC_W = [(1, EF), (1, EF), (EF, L), _B, _W, _B, _B, _B]

_enc_node = _mlp_call(_enc_node_body, Np, 2560, [(NF, F32)],
                      _F, [_B, _B, _W, _B, _W, _B, _B, _B])
_edge_first = _mlp_call(_edge_first_body, Ep, 4096, _F * 2 + [(EF, F32)],
                        _F * 2, _ENC_W + _EDGE_W)
_edge_mid = _mlp_call(_edge_mid_body, Ep, 4096, _F * 3,
                      _F * 2, _EDGE_W)
_edge_last = _mlp_call(_edge_last_body, Ep, 4096, _F * 3,
                       _F, _EDGE_W)
_node_step = _mlp_call(_node_last_body, Np, 2560, _F * 3,
                       _F, [_W, _W, _B, _W, _B, _B, _B])
_decoder = _mlp_call(_decoder_body, Np, 2560, _F,
                     _F, [_W, _B, _W, _B, _B])


def kernel(node_features, mesh_edge_features, senders, receivers, params):
    p = params
    nf = jnp.pad(node_features.reshape(N, NF), ((0, Np - N), (0, 0)))
    ef = jnp.pad(mesh_edge_features.reshape(E, EF), ((0, Ep - E), (0, 0)))
    s_pad = jnp.pad(senders.astype(jnp.int32), (0, Ep - E),
                    constant_values=Np - 1)
    r_pad = jnp.pad(receivers.astype(jnp.int32), (0, Ep - E),
                    constant_values=Np - 1)
    s_idx3 = s_pad.reshape(NT, ECH_T, CH)
    r_idx3 = r_pad.reshape(NT, ECH_T, CH)

    def row(v):
        return v.reshape(1, -1)

    def ew1(k):
        w1 = p["blk%d_e_W1" % k]
        return w1[:L], w1[L:2 * L], w1[2 * L:]

    def edge_w(k):
        pe = "blk%d_e" % k
        return (*ew1(k), row(p[pe + "_b1"]), p[pe + "_W2"],
                row(p[pe + "_b2"]), row(p[pe + "_g"]), row(p[pe + "_be"]))

    def node_w(k):
        pn = "blk%d_n" % k
        w1n = p[pn + "_W1"]
        return (w1n[:L], w1n[L:], row(p[pn + "_b1"]), p[pn + "_W2"],
                row(p[pn + "_b2"]), row(p[pn + "_g"]), row(p[pn + "_be"]))

    (node_lat,) = _enc_node(
        nf, row(p["node_mean"]), row(p["node_std"]),
        p["enc_n_W1"], row(p["enc_n_b1"]), p["enc_n_W2"], row(p["enc_n_b2"]),
        row(p["enc_n_g"]), row(p["enc_n_be"]))

    sc_gather, sc_scatter = _sc_kernels()
    zeros_rows = jnp.zeros((CH, L), F32)

    edge_lat = None
    for k in range(STEPS):
        gs, gr = sc_gather(node_lat, s_idx3, r_idx3)
        if k == 0:
            new_e, edge_lat = _edge_first(
                gs, gr, ef, row(p["edge_mean"]), row(p["edge_std"]),
                p["enc_e_W1"], row(p["enc_e_b1"]), p["enc_e_W2"],
                row(p["enc_e_b2"]), row(p["enc_e_g"]), row(p["enc_e_be"]),
                *edge_w(k))
        elif k < STEPS - 1:
            new_e, edge_lat = _edge_mid(gs, gr, edge_lat, *edge_w(k))
        else:
            (new_e,) = _edge_last(gs, gr, edge_lat, *edge_w(k))
        aggr = sc_scatter(new_e, r_idx3, zeros_rows)
        (node_lat,) = _node_step(node_lat, aggr[0], aggr[1], *node_w(k))

    w1p = jnp.zeros((L, L), F32).at[:, :8].set(p["dec_W1"].T)
    b1p = jnp.zeros((1, L), F32).at[0, :8].set(p["dec_b1"])
    w2p = jnp.zeros((L, L), F32).at[:8, :TW].set(p["dec_W2"].T)
    b2p = jnp.zeros((1, L), F32).at[0, :TW].set(p["dec_b2"])
    dt = jnp.zeros((1, L), F32).at[0, :TW].set(
        jnp.arange(1, TW + 1, dtype=F32))
    (dec,) = _decoder(node_lat, w1p, b1p, w2p, b2p, dt)
    return dec[:N, :TW].T[:, :, None]


# speedup vs baseline: 5.5744x; 1.0087x over previous
import functools

import jax
import jax.numpy as jnp
from jax import lax
from jax.experimental import pallas as pl
from jax.experimental.pallas import tpu as pltpu
from jax.experimental.pallas import tpu_sc as plsc

N, E, NF, EF, L, TW, OUT, STEPS = 10000, 320000, 128, 4, 128, 5, 1, 5

NC, NS = 2, 16
NT = NC * NS
CH = 128
Np = 10240
ECH_T = 80
EPT = ECH_T * CH
Ep = NT * EPT
NPT = Np // NS
NCH_T = NPT // CH

F32 = jnp.float32
BF16 = jnp.bfloat16


@functools.cache
def _sc_kernels():
    mesh = plsc.VectorSubcoreMesh(
        core_axis_name="c", subcore_axis_name="s",
        num_cores=NC, num_subcores=NS)

    @functools.partial(
        pl.kernel,
        out_type=[jax.ShapeDtypeStruct((Ep, L), F32),
                  jax.ShapeDtypeStruct((Ep, L), F32)],
        mesh=mesh,
        scratch_types=[
            pltpu.VMEM((ECH_T, CH), jnp.int32),
            pltpu.VMEM((2, CH, L), F32),
            pltpu.VMEM_SHARED((Np, L), F32),
            pltpu.SemaphoreType.DMA,
            pltpu.SemaphoreType.DMA,
            pltpu.SemaphoreType.DMA,
            pltpu.SemaphoreType.DMA,
        ],
    )
    def sc_gather(nodes, s_idx, r_idx, gs_out, gr_out,
                  idx, bufs, table, g0, g1, w0, w1):
        cid = lax.axis_index("c")
        sid = lax.axis_index("s")
        wid = sid * NC + cid
        base0 = wid * EPT
        gsem = (g0, g1)
        wsem = (w0, w1)

        row0 = sid * NPT

        def stage(t, carry):
            r0 = row0 + t * CH
            pltpu.sync_copy(nodes.at[pl.ds(r0, CH)], bufs.at[0])
            pltpu.sync_copy(bufs.at[0], table.at[pl.ds(r0, CH)])
            return carry

        lax.fori_loop(0, NCH_T, stage, 0)
        pltpu.sync_copy(s_idx.at[wid], idx)
        plsc.subcore_barrier()

        NH = ECH_T // 2

        def run_phase(out_ref):

            def fire_g(slot, j):
                pltpu.async_copy(table.at[idx.at[j]], bufs.at[slot],
                                 gsem[slot])

            def wait_g(slot):
                pltpu.make_async_copy(table.at[idx.at[0]], bufs.at[slot],
                                      gsem[slot]).wait()

            def fire_w(slot, j):
                pltpu.async_copy(bufs.at[slot],
                                 out_ref.at[pl.ds(base0 + j * CH, CH)],
                                 wsem[slot])

            def wait_w(slot):
                pltpu.make_async_copy(bufs.at[slot],
                                      out_ref.at[pl.ds(0, CH)],
                                      wsem[slot]).wait()

            fire_g(0, 0)

            def body(jj, carry):
                j0 = 2 * jj
                pl.when(jj > 0)(lambda: wait_w(1))
                fire_g(1, j0 + 1)
                wait_g(0)
                fire_w(0, j0)

                def refill_a():
                    wait_w(0)
                    fire_g(0, j0 + 2)

                pl.when(jj < NH - 1)(refill_a)
                wait_g(1)
                fire_w(1, j0 + 1)
                return carry

            lax.fori_loop(0, NH, body, 0)
            wait_w(0)
            wait_w(1)

        run_phase(gs_out)
        pltpu.sync_copy(r_idx.at[wid], idx)
        run_phase(gr_out)

    @functools.partial(
        pl.kernel,
        out_type=jax.ShapeDtypeStruct((NC, Np, L), F32),
        mesh=mesh,
        scratch_types=[
            pltpu.VMEM((ECH_T, CH), jnp.int32),
            pltpu.VMEM((2, CH, L), F32),
            pltpu.VMEM_SHARED((Np, L), F32),
            pltpu.SemaphoreType.DMA,
            pltpu.SemaphoreType.DMA,
        ],
    )
    def sc_scatter(msgs, r_idx, zeros_rows, aggr_out, idx_v, rows, acc,
                   l0, l1):
        cid = lax.axis_index("c")
        sid = lax.axis_index("s")
        wid = sid * NC + cid

        pltpu.sync_copy(zeros_rows, rows.at[0])
        row0 = sid * NPT

        def zbody(t, carry):
            pltpu.sync_copy(rows.at[0], acc.at[pl.ds(row0 + t * CH, CH)])
            return carry

        lax.fori_loop(0, NCH_T, zbody, 0)
        pltpu.sync_copy(r_idx.at[wid], idx_v)
        plsc.subcore_barrier()

        base0 = wid * EPT
        lsem = (l0, l1)

        def fire_l(slot, j):
            pltpu.async_copy(msgs.at[pl.ds(base0 + j * CH, CH)],
                             rows.at[slot], lsem[slot])

        def wait_l(slot):
            pltpu.make_async_copy(msgs.at[pl.ds(0, CH)], rows.at[slot],
                                  lsem[slot]).wait()

        def scat(slot, j):
            pltpu.sync_copy(rows.at[slot], acc.at[idx_v.at[j]], add=True)

        fire_l(0, 0)
        NH = ECH_T // 2

        def body(jj, carry):
            j0 = 2 * jj
            fire_l(1, j0 + 1)
            wait_l(0)
            scat(0, j0)
            pl.when(jj < NH - 1)(lambda: fire_l(0, j0 + 2))
            wait_l(1)
            scat(1, j0 + 1)
            return carry

        lax.fori_loop(0, NH, body, 0)
        plsc.subcore_barrier()

        def dbody(t, carry):
            r0 = row0 + t * CH
            pltpu.sync_copy(acc.at[pl.ds(r0, CH)], rows.at[0])
            pltpu.sync_copy(rows.at[0], aggr_out.at[cid, pl.ds(r0, CH)])
            return carry

        lax.fori_loop(0, NCH_T, dbody, 0)

    return sc_gather, sc_scatter


def _ln(h, g, be):
    mu = jnp.mean(h, axis=-1, keepdims=True)
    var = jnp.mean((h - mu) ** 2, axis=-1, keepdims=True)
    return (h - mu) * lax.rsqrt(var + 1e-5) * g + be


def _row_spec(br, d):
    return pl.BlockSpec((br, d), lambda i: (i, 0))


def _full_spec(shape):
    return pl.BlockSpec(shape, lambda i: tuple(0 for _ in shape))


def _dot(a, b):
    return jnp.dot(a, b, preferred_element_type=F32)


def _enc_node_body(x, m, s, w1, b1, w2, b2, g, be, o):
    xn = (x[...] - m[...]) / (s[...] + 1e-8)
    h = jnp.maximum(_dot(xn, w1[...]) + b1[...], 0.0)
    h = jnp.maximum(_dot(h, w2[...]) + b2[...], 0.0)
    o[...] = _ln(h, g[...], be[...])


def _enc_edge(x, m, s, w1, b1, w2, b2, g, be):
    xn = (x[...] - m[...]) / (s[...] + 1e-8)
    h = b1[...]
    for f in range(EF):
        h = h + xn[:, f:f + 1] * w1[f:f + 1, :]
    h = jnp.maximum(h, 0.0)
    h = jnp.maximum(_dot(h, w2[...]) + b2[...], 0.0)
    return _ln(h, g[...], be[...])


def _edge_core(gs, gr, el, w1a, w1b, w1c, b1, w2, b2, g, be):
    x = (_dot(gs, w1a[...]) + _dot(gr, w1b[...])
         + _dot(el, w1c[...]) + b1[...])
    h = jnp.maximum(x, 0.0)
    h = jnp.maximum(_dot(h, w2[...]) + b2[...], 0.0)
    return _ln(h, g[...], be[...])


def _edge_first_body(gs, gr, ef, em, es, ew1, eb1, ew2, eb2, eg, ebe,
                     w1a, w1b, w1c, b1, w2, b2, g, be, ne, nl):
    el = _enc_edge(ef, em, es, ew1, eb1, ew2, eb2, eg, ebe)
    ln = _edge_core(gs[...], gr[...], el, w1a, w1b, w1c, b1, w2, b2, g, be)
    ne[...] = ln
    nl[...] = el + ln


def _edge_mid_body(gs, gr, el, w1a, w1b, w1c, b1, w2, b2, g, be, ne, nl):
    ln = _edge_core(gs[...], gr[...], el[...], w1a, w1b, w1c, b1, w2, b2,
                    g, be)
    ne[...] = ln
    nl[...] = el[...] + ln


def _edge_last_body(gs, gr, el, w1a, w1b, w1c, b1, w2, b2, g, be, ne):
    ne[...] = _edge_core(gs[...], gr[...], el[...], w1a, w1b, w1c, b1, w2, b2,
                         g, be)


def _node_core(nl, a0, a1, w1a, w1b, b1, w2, b2, g, be):
    x = (_dot(nl[...], w1a[...]) + _dot(a0[...] + a1[...], w1b[...]) + b1[...])
    h = jnp.maximum(x, 0.0)
    h = jnp.maximum(_dot(h, w2[...]) + b2[...], 0.0)
    return _ln(h, g[...], be[...]) + nl[...]


def _node_last_body(nl, a0, a1, w1a, w1b, b1, w2, b2, g, be, o):
    o[...] = _node_core(nl, a0, a1, w1a, w1b, b1, w2, b2, g, be)


def _decoder_body(x, w1p, b1p, w2p, b2p, dt, o):
    h = _dot(x[...], w1p[...]) + b1p[...]
    h = h * jax.nn.sigmoid(h)
    o[...] = (_dot(h, w2p[...]) + b2p[...]) * dt[...]


def _mlp_call(body, rows, br, in_feats, outs, weight_shapes):
    grid = rows // br
    in_specs = ([_row_spec(br, d) for d, _ in in_feats]
                + [_full_spec(s) for s in weight_shapes])
    return pl.pallas_call(
        body,
        grid=(grid,),
        in_specs=in_specs,
        out_specs=[_row_spec(br, d) for d, _ in outs],
        out_shape=[jax.ShapeDtypeStruct((rows, d), dt) for d, dt in outs],
        compiler_params=pltpu.CompilerParams(
            dimension_semantics=("parallel",)),
    )


_W = (L, L)
_B = (1, L)
_F = [(L, F32)]
_H = [(L, BF16)]
_EDGE_W = [_W, _W, _W, _B, _W, _B, _B, _B]
_ENC_W = [(1, EF), (1, EF), (EF, L), _B, _W, _B, _B, _B]

_enc_node = _mlp_call(_enc_node_body, Np, 2560, [(NF, F32)],
                      _F, [_B, _B, _W, _B, _W, _B, _B, _B])
_edge_first = _mlp_call(_edge_first_body, Ep, 8192, _F * 2 + [(EF, F32)],
                        _F * 2, _ENC_W + _EDGE_W)
_edge_mid = _mlp_call(_edge_mid_body, Ep, 8192, _F * 3,
                      _F * 2, _EDGE_W)
_edge_last = _mlp_call(_edge_last_body, Ep, 8192, _F * 3,
                       _F, _EDGE_W)
_node_step = _mlp_call(_node_last_body, Np, 2560, _F * 3,
                       _F, [_W, _W, _B, _W, _B, _B, _B])
_decoder = _mlp_call(_decoder_body, Np, 2560, _F,
                     _F, [_W, _B, _W, _B, _B])


def kernel(node_features, mesh_edge_features, senders, receivers, params):
    p = params
    nf = jnp.pad(node_features.reshape(N, NF), ((0, Np - N), (0, 0)))
    ef = jnp.pad(mesh_edge_features.reshape(E, EF), ((0, Ep - E), (0, 0)))
    s_pad = jnp.pad(senders.astype(jnp.int32), (0, Ep - E),
                    constant_values=Np - 1)
    r_pad = jnp.pad(receivers.astype(jnp.int32), (0, Ep - E),
                    constant_values=Np - 1)
    s_idx3 = s_pad.reshape(NT, ECH_T, CH)
    r_idx3 = r_pad.reshape(NT, ECH_T, CH)

    def row(v):
        return v.reshape(1, -1)

    def ew1(k):
        w1 = p["blk%d_e_W1" % k]
        return w1[:L], w1[L:2 * L], w1[2 * L:]

    def edge_w(k):
        pe = "blk%d_e" % k
        return (*ew1(k), row(p[pe + "_b1"]), p[pe + "_W2"],
                row(p[pe + "_b2"]), row(p[pe + "_g"]), row(p[pe + "_be"]))

    def node_w(k):
        pn = "blk%d_n" % k
        w1n = p[pn + "_W1"]
        return (w1n[:L], w1n[L:], row(p[pn + "_b1"]), p[pn + "_W2"],
                row(p[pn + "_b2"]), row(p[pn + "_g"]), row(p[pn + "_be"]))

    (node_lat,) = _enc_node(
        nf, row(p["node_mean"]), row(p["node_std"]),
        p["enc_n_W1"], row(p["enc_n_b1"]), p["enc_n_W2"], row(p["enc_n_b2"]),
        row(p["enc_n_g"]), row(p["enc_n_be"]))

    sc_gather, sc_scatter = _sc_kernels()
    zeros_rows = jnp.zeros((CH, L), F32)

    edge_lat = None
    for k in range(STEPS):
        gs, gr = sc_gather(node_lat, s_idx3, r_idx3)
        if k == 0:
            new_e, edge_lat = _edge_first(
                gs, gr, ef, row(p["edge_mean"]), row(p["edge_std"]),
                p["enc_e_W1"], row(p["enc_e_b1"]), p["enc_e_W2"],
                row(p["enc_e_b2"]), row(p["enc_e_g"]), row(p["enc_e_be"]),
                *edge_w(k))
        elif k < STEPS - 1:
            new_e, edge_lat = _edge_mid(gs, gr, edge_lat, *edge_w(k))
        else:
            (new_e,) = _edge_last(gs, gr, edge_lat, *edge_w(k))
        aggr = sc_scatter(new_e, r_idx3, zeros_rows)
        (node_lat,) = _node_step(node_lat, aggr[0], aggr[1], *node_w(k))

    w1p = jnp.zeros((L, L), F32).at[:, :8].set(p["dec_W1"].T)
    b1p = jnp.zeros((1, L), F32).at[0, :8].set(p["dec_b1"])
    w2p = jnp.zeros((L, L), F32).at[:8, :TW].set(p["dec_W2"].T)
    b2p = jnp.zeros((1, L), F32).at[0, :TW].set(p["dec_b2"])
    dt = jnp.zeros((1, L), F32).at[0, :TW].set(
        jnp.arange(1, TW + 1, dtype=F32))
    (dec,) = _decoder(node_lat, w1p, b1p, w2p, b2p, dt)
    return dec[:N, :TW].T[:, :, None]


# speedup vs baseline: 5.5778x; 1.0006x over previous
import functools

import jax
import jax.numpy as jnp
from jax import lax
from jax.experimental import pallas as pl
from jax.experimental.pallas import tpu as pltpu
from jax.experimental.pallas import tpu_sc as plsc

N, E, NF, EF, L, TW, OUT, STEPS = 10000, 320000, 128, 4, 128, 5, 1, 5

NC, NS = 2, 16
NT = NC * NS
CH = 128
Np = 10240
ECH_T = 80
EPT = ECH_T * CH
Ep = NT * EPT
NPT = Np // NS
NCH_T = NPT // CH

F32 = jnp.float32


@functools.cache
def _sc_kernels():
    mesh = plsc.VectorSubcoreMesh(
        core_axis_name="c", subcore_axis_name="s",
        num_cores=NC, num_subcores=NS)

    @functools.partial(
        pl.kernel,
        out_type=[jax.ShapeDtypeStruct((Ep, L), F32),
                  jax.ShapeDtypeStruct((Ep, L), F32)],
        mesh=mesh,
        scratch_types=[
            pltpu.VMEM((ECH_T, CH), jnp.int32),
            pltpu.VMEM((2, CH, L), F32),
            pltpu.VMEM_SHARED((Np, L), F32),
            pltpu.SemaphoreType.DMA,
            pltpu.SemaphoreType.DMA,
            pltpu.SemaphoreType.DMA,
            pltpu.SemaphoreType.DMA,
        ],
    )
    def sc_gather(nodes, s_idx, r_idx, gs_out, gr_out,
                  idx, bufs, table, g0, g1, w0, w1):
        cid = lax.axis_index("c")
        sid = lax.axis_index("s")
        wid = sid * NC + cid
        base0 = wid * EPT
        gsem = (g0, g1)
        wsem = (w0, w1)

        row0 = sid * NPT

        def stage(t, carry):
            r0 = row0 + t * CH
            pltpu.sync_copy(nodes.at[pl.ds(r0, CH)], bufs.at[0])
            pltpu.sync_copy(bufs.at[0], table.at[pl.ds(r0, CH)])
            return carry

        lax.fori_loop(0, NCH_T, stage, 0)
        pltpu.sync_copy(s_idx.at[wid], idx)
        plsc.subcore_barrier()

        NH = ECH_T // 2

        def run_phase(out_ref):

            def fire_g(slot, j):
                pltpu.async_copy(table.at[idx.at[j]], bufs.at[slot],
                                 gsem[slot])

            def wait_g(slot):
                pltpu.make_async_copy(table.at[idx.at[0]], bufs.at[slot],
                                      gsem[slot]).wait()

            def fire_w(slot, j):
                pltpu.async_copy(bufs.at[slot],
                                 out_ref.at[pl.ds(base0 + j * CH, CH)],
                                 wsem[slot])

            def wait_w(slot):
                pltpu.make_async_copy(bufs.at[slot],
                                      out_ref.at[pl.ds(0, CH)],
                                      wsem[slot]).wait()

            fire_g(0, 0)

            def body(jj, carry):
                j0 = 2 * jj
                pl.when(jj > 0)(lambda: wait_w(1))
                fire_g(1, j0 + 1)
                wait_g(0)
                fire_w(0, j0)

                def refill_a():
                    wait_w(0)
                    fire_g(0, j0 + 2)

                pl.when(jj < NH - 1)(refill_a)
                wait_g(1)
                fire_w(1, j0 + 1)
                return carry

            lax.fori_loop(0, NH, body, 0)
            wait_w(0)
            wait_w(1)

        run_phase(gs_out)
        pltpu.sync_copy(r_idx.at[wid], idx)
        run_phase(gr_out)

    @functools.partial(
        pl.kernel,
        out_type=jax.ShapeDtypeStruct((NC, Np, L), F32),
        mesh=mesh,
        scratch_types=[
            pltpu.VMEM((ECH_T, CH), jnp.int32),
            pltpu.VMEM((2, CH, L), F32),
            pltpu.VMEM_SHARED((Np, L), F32),
            pltpu.SemaphoreType.DMA,
            pltpu.SemaphoreType.DMA,
        ],
    )
    def sc_scatter(msgs, r_idx, zeros_rows, aggr_out, idx_v, rows, acc,
                   l0, l1):
        cid = lax.axis_index("c")
        sid = lax.axis_index("s")
        wid = sid * NC + cid

        pltpu.sync_copy(zeros_rows, rows.at[0])
        row0 = sid * NPT

        def zbody(t, carry):
            pltpu.sync_copy(rows.at[0], acc.at[pl.ds(row0 + t * CH, CH)])
            return carry

        lax.fori_loop(0, NCH_T, zbody, 0)
        pltpu.sync_copy(r_idx.at[wid], idx_v)
        plsc.subcore_barrier()

        base0 = wid * EPT
        lsem = (l0, l1)

        def fire_l(slot, j):
            pltpu.async_copy(msgs.at[pl.ds(base0 + j * CH, CH)],
                             rows.at[slot], lsem[slot])

        def wait_l(slot):
            pltpu.make_async_copy(msgs.at[pl.ds(0, CH)], rows.at[slot],
                                  lsem[slot]).wait()

        def scat(slot, j):
            pltpu.sync_copy(rows.at[slot], acc.at[idx_v.at[j]], add=True)

        fire_l(0, 0)
        NH = ECH_T // 2

        def body(jj, carry):
            j0 = 2 * jj
            fire_l(1, j0 + 1)
            wait_l(0)
            scat(0, j0)
            pl.when(jj < NH - 1)(lambda: fire_l(0, j0 + 2))
            wait_l(1)
            scat(1, j0 + 1)
            return carry

        lax.fori_loop(0, NH, body, 0)
        plsc.subcore_barrier()

        def dbody(t, carry):
            r0 = row0 + t * CH
            pltpu.sync_copy(acc.at[pl.ds(r0, CH)], rows.at[0])
            pltpu.sync_copy(rows.at[0], aggr_out.at[cid, pl.ds(r0, CH)])
            return carry

        lax.fori_loop(0, NCH_T, dbody, 0)

    return sc_gather, sc_scatter


def _ln(h, g, be):
    mu = jnp.mean(h, axis=-1, keepdims=True)
    var = jnp.mean((h - mu) ** 2, axis=-1, keepdims=True)
    return (h - mu) * lax.rsqrt(var + 1e-5) * g + be


def _row_spec(br, d):
    return pl.BlockSpec((br, d), lambda i: (i, 0))


def _full_spec(shape):
    return pl.BlockSpec(shape, lambda i: tuple(0 for _ in shape))


def _dot(a, b):
    return jnp.dot(a, b, preferred_element_type=F32)


def _enc_node_body(x, m, s, w1, b1, w2, b2, g, be, o):
    xn = (x[...] - m[...]) / (s[...] + 1e-8)
    h = jnp.maximum(_dot(xn, w1[...]) + b1[...], 0.0)
    h = jnp.maximum(_dot(h, w2[...]) + b2[...], 0.0)
    o[...] = _ln(h, g[...], be[...])


def _enc_edge(x, m, s, w1, b1, w2, b2, g, be):
    xn = (x[...] - m[...]) / (s[...] + 1e-8)
    h = b1[...]
    for f in range(EF):
        h = h + xn[:, f:f + 1] * w1[f:f + 1, :]
    h = jnp.maximum(h, 0.0)
    h = jnp.maximum(_dot(h, w2[...]) + b2[...], 0.0)
    return _ln(h, g[...], be[...])


def _edge_core(gs, gr, el, w1a, w1b, w1c, b1, w2, b2, g, be):
    x = (_dot(gs, w1a[...]) + _dot(gr, w1b[...])
         + _dot(el, w1c[...]) + b1[...])
    h = jnp.maximum(x, 0.0)
    h = jnp.maximum(_dot(h, w2[...]) + b2[...], 0.0)
    return _ln(h, g[...], be[...])


def _edge_first_body(gs, gr, ef, em, es, ew1, eb1, ew2, eb2, eg, ebe,
                     w1a, w1b, w1c, b1, w2, b2, g, be, ne, nl):
    el = _enc_edge(ef, em, es, ew1, eb1, ew2, eb2, eg, ebe)
    ln = _edge_core(gs[...], gr[...], el, w1a, w1b, w1c, b1, w2, b2, g, be)
    ne[...] = ln
    nl[...] = el + ln


def _edge_mid_body(gs, gr, el, w1a, w1b, w1c, b1, w2, b2, g, be, ne, nl):
    ln = _edge_core(gs[...], gr[...], el[...], w1a, w1b, w1c, b1, w2, b2,
                    g, be)
    ne[...] = ln
    nl[...] = el[...] + ln


def _edge_last_body(gs, gr, el, w1a, w1b, w1c, b1, w2, b2, g, be, ne):
    ne[...] = _edge_core(gs[...], gr[...], el[...], w1a, w1b, w1c, b1, w2, b2,
                         g, be)


def _node_core(nl, a0, a1, w1a, w1b, b1, w2, b2, g, be):
    x = (_dot(nl[...], w1a[...]) + _dot(a0[...] + a1[...], w1b[...]) + b1[...])
    h = jnp.maximum(x, 0.0)
    h = jnp.maximum(_dot(h, w2[...]) + b2[...], 0.0)
    return _ln(h, g[...], be[...]) + nl[...]


def _node_last_body(nl, a0, a1, w1a, w1b, b1, w2, b2, g, be, o):
    o[...] = _node_core(nl, a0, a1, w1a, w1b, b1, w2, b2, g, be)


def _decoder_body(x, w1p, b1p, w2p, b2p, dt, o):
    h = _dot(x[...], w1p[...]) + b1p[...]
    h = h * jax.nn.sigmoid(h)
    o[...] = (_dot(h, w2p[...]) + b2p[...]) * dt[...]


def _mlp_call(body, rows, br, in_feats, outs, weight_shapes):
    grid = rows // br
    in_specs = ([_row_spec(br, d) for d, _ in in_feats]
                + [_full_spec(s) for s in weight_shapes])
    return pl.pallas_call(
        body,
        grid=(grid,),
        in_specs=in_specs,
        out_specs=[_row_spec(br, d) for d, _ in outs],
        out_shape=[jax.ShapeDtypeStruct((rows, d), dt) for d, dt in outs],
        compiler_params=pltpu.CompilerParams(
            dimension_semantics=("parallel",)),
    )


_W = (L, L)
_B = (1, L)
_F = [(L, F32)]
_EDGE_W = [_W, _W, _W, _B, _W, _B, _B, _B]
_ENC_W = [(1, EF), (1, EF), (EF, L), _B, _W, _B, _B, _B]

_enc_node = _mlp_call(_enc_node_body, Np, 2560, [(NF, F32)],
                      _F, [_B, _B, _W, _B, _W, _B, _B, _B])
_edge_first = _mlp_call(_edge_first_body, Ep, 8192, _F * 2 + [(EF, F32)],
                        _F * 2, _ENC_W + _EDGE_W)
_edge_mid = _mlp_call(_edge_mid_body, Ep, 8192, _F * 3,
                      _F * 2, _EDGE_W)
_edge_last = _mlp_call(_edge_last_body, Ep, 8192, _F * 3,
                       _F, _EDGE_W)
_node_step = _mlp_call(_node_last_body, Np, 2560, _F * 3,
                       _F, [_W, _W, _B, _W, _B, _B, _B])
_decoder = _mlp_call(_decoder_body, Np, 2560, _F,
                     _F, [_W, _B, _W, _B, _B])


def kernel(node_features, mesh_edge_features, senders, receivers, params):
    p = params
    nf = jnp.pad(node_features.reshape(N, NF), ((0, Np - N), (0, 0)))
    ef = jnp.pad(mesh_edge_features.reshape(E, EF), ((0, Ep - E), (0, 0)))
    s_pad = jnp.pad(senders.astype(jnp.int32), (0, Ep - E),
                    constant_values=Np - 1)
    r_pad = jnp.pad(receivers.astype(jnp.int32), (0, Ep - E),
                    constant_values=Np - 1)
    s_idx3 = s_pad.reshape(NT, ECH_T, CH)
    r_idx3 = r_pad.reshape(NT, ECH_T, CH)

    def row(v):
        return v.reshape(1, -1)

    def ew1(k):
        w1 = p["blk%d_e_W1" % k]
        return w1[:L], w1[L:2 * L], w1[2 * L:]

    def edge_w(k):
        pe = "blk%d_e" % k
        return (*ew1(k), row(p[pe + "_b1"]), p[pe + "_W2"],
                row(p[pe + "_b2"]), row(p[pe + "_g"]), row(p[pe + "_be"]))

    def node_w(k):
        pn = "blk%d_n" % k
        w1n = p[pn + "_W1"]
        return (w1n[:L], w1n[L:], row(p[pn + "_b1"]), p[pn + "_W2"],
                row(p[pn + "_b2"]), row(p[pn + "_g"]), row(p[pn + "_be"]))

    (node_lat,) = _enc_node(
        nf, row(p["node_mean"]), row(p["node_std"]),
        p["enc_n_W1"], row(p["enc_n_b1"]), p["enc_n_W2"], row(p["enc_n_b2"]),
        row(p["enc_n_g"]), row(p["enc_n_be"]))

    sc_gather, sc_scatter = _sc_kernels()
    zeros_rows = jnp.zeros((CH, L), F32)

    edge_lat = None
    for k in range(STEPS):
        gs, gr = sc_gather(node_lat, s_idx3, r_idx3)
        if k == 0:
            new_e, edge_lat = _edge_first(
                gs, gr, ef, row(p["edge_mean"]), row(p["edge_std"]),
                p["enc_e_W1"], row(p["enc_e_b1"]), p["enc_e_W2"],
                row(p["enc_e_b2"]), row(p["enc_e_g"]), row(p["enc_e_be"]),
                *edge_w(k))
        elif k < STEPS - 1:
            new_e, edge_lat = _edge_mid(gs, gr, edge_lat, *edge_w(k))
        else:
            (new_e,) = _edge_last(gs, gr, edge_lat, *edge_w(k))
        aggr = sc_scatter(new_e, r_idx3, zeros_rows)
        (node_lat,) = _node_step(node_lat, aggr[0], aggr[1], *node_w(k))

    w1p = jnp.zeros((L, L), F32).at[:, :8].set(p["dec_W1"].T)
    b1p = jnp.zeros((1, L), F32).at[0, :8].set(p["dec_b1"])
    w2p = jnp.zeros((L, L), F32).at[:8, :TW].set(p["dec_W2"].T)
    b2p = jnp.zeros((1, L), F32).at[0, :TW].set(p["dec_b2"])
    dt = jnp.zeros((1, L), F32).at[0, :TW].set(
        jnp.arange(1, TW + 1, dtype=F32))
    (dec,) = _decoder(node_lat, w1p, b1p, w2p, b2p, dt)
    return dec[:N, :TW].T[:, :, None]
